# baseline (speedup 1.0000x reference)
import functools

import jax
import jax.numpy as jnp
from jax import lax
from jax.experimental import pallas as pl
from jax.experimental.pallas import tpu as pltpu
from jax.experimental.pallas import tpu_sc as plsc

_NC = 2
_NS = 16
_NW = _NC * _NS
_R = 8
_NBUF = 2


@functools.lru_cache(maxsize=None)
def _make_gather(batch, fields, d):
    rows_w = batch // _NW
    nchunks = rows_w // _R
    mesh = plsc.VectorSubcoreMesh(core_axis_name="c", subcore_axis_name="s")

    @functools.partial(
        pl.kernel,
        mesh=mesh,
        out_type=jax.ShapeDtypeStruct((batch, fields, d), jnp.float32),
        scratch_types=[
            pltpu.VMEM((rows_w, fields), jnp.int32),
            pltpu.VMEM((_NBUF, _R, fields, d), jnp.float32),
        ]
        + [pltpu.SemaphoreType.DMA] * (2 * _NBUF),
        compiler_params=pltpu.CompilerParams(use_tc_tiling_on_sc=True),
    )
    def k(obs_hbm, table_hbm, out_hbm, idx_v, rows_v, *sems):
        semg = sems[:_NBUF]
        semw = sems[_NBUF:]
        wid = lax.axis_index("s") * _NC + lax.axis_index("c")
        row0 = wid * rows_w
        pltpu.sync_copy(obs_hbm.at[pl.ds(row0, rows_w)], idx_v)

        def fire(ch, b):
            def f(r, c):
                pltpu.make_async_copy(
                    table_hbm.at[idx_v.at[ch * _R + r]],
                    rows_v.at[b, r],
                    semg[b],
                ).start()
                return c

            lax.fori_loop(0, _R, f, 0)

        def drain(b):
            def f(r, c):
                pltpu.make_async_copy(
                    table_hbm.at[idx_v.at[0]], rows_v.at[b, r], semg[b]
                ).wait()
                return c

            lax.fori_loop(0, _R, f, 0)

        def write_start(ch, b):
            pltpu.make_async_copy(
                rows_v.at[b], out_hbm.at[pl.ds(row0 + ch * _R, _R)], semw[b]
            ).start()

        def write_wait(b):
            pltpu.make_async_copy(
                rows_v.at[b], out_hbm.at[pl.ds(row0, _R)], semw[b]
            ).wait()

        fire(0, 0)

        def body(i, carry):
            for b in range(_NBUF):
                ch = _NBUF * i + b
                nb = (b + 1) % _NBUF
                cnext = ch + 1

                @pl.when(cnext < nchunks)
                def _():
                    @pl.when(cnext >= _NBUF)
                    def _():
                        write_wait(nb)

                    fire(cnext, nb)

                drain(b)
                write_start(ch, b)
            return carry

        lax.fori_loop(0, nchunks // _NBUF, body, 0)
        for b in range(_NBUF):
            write_wait(b)

    return k


def kernel(obs, emb_table):
    batch, fields = obs.shape
    n_cat, d = emb_table.shape
    return _make_gather(batch, fields, d)(obs.astype(jnp.int32), emb_table)


# speedup vs baseline: 2.0668x; 2.0668x over previous
import functools

import jax
import jax.numpy as jnp
from jax import lax
from jax.experimental import pallas as pl
from jax.experimental.pallas import tpu as pltpu
from jax.experimental.pallas import tpu_sc as plsc

_NC = 2
_NS = 16
_NW = _NC * _NS
_C = 128
_NBUF = 4
_DEPTH = 3


@functools.lru_cache(maxsize=None)
def _make_gather(batch, fields, d):
    rows_w = batch // _NW
    nsub = rows_w // _C
    nchunk = fields * nsub
    mesh = plsc.VectorSubcoreMesh(core_axis_name="c", subcore_axis_name="s")

    @functools.partial(
        pl.kernel,
        mesh=mesh,
        out_type=jax.ShapeDtypeStruct((fields, batch, d), jnp.float32),
        scratch_types=[
            pltpu.VMEM((fields, rows_w), jnp.int32),
            pltpu.VMEM((_NBUF, _C, d), jnp.float32),
        ]
        + [pltpu.SemaphoreType.DMA] * (2 * _NBUF),
    )
    def k(obs_t_hbm, table_hbm, out_hbm, idx_v, rows_v, *sems):
        semg = sems[:_NBUF]
        semw = sems[_NBUF:]
        wid = lax.axis_index("s") * _NC + lax.axis_index("c")
        row0 = wid * rows_w
        pltpu.sync_copy(obs_t_hbm.at[:, pl.ds(row0, rows_w)], idx_v)

        def gather_start(c, b):
            f = c // nsub
            sub = c - f * nsub
            pltpu.make_async_copy(
                table_hbm.at[idx_v.at[f, pl.ds(sub * _C, _C)]],
                rows_v.at[b],
                semg[b],
            ).start()

        def gather_wait(b):
            pltpu.make_async_copy(
                table_hbm.at[idx_v.at[0, pl.ds(0, _C)]], rows_v.at[b], semg[b]
            ).wait()

        def write_start(c, b):
            f = c // nsub
            sub = c - f * nsub
            pltpu.make_async_copy(
                rows_v.at[b],
                out_hbm.at[f, pl.ds(row0 + sub * _C, _C)],
                semw[b],
            ).start()

        def write_wait(b):
            pltpu.make_async_copy(
                rows_v.at[b], out_hbm.at[0, pl.ds(row0, _C)], semw[b]
            ).wait()

        for c in range(_DEPTH):
            gather_start(c, c)

        def body(i, carry):
            for b in range(_NBUF):
                c = _NBUF * i + b
                gather_wait(b)
                write_start(c, b)
                nb = (b + _DEPTH) % _NBUF
                cn = c + _DEPTH

                @pl.when(cn < nchunk)
                def _():
                    @pl.when(cn - _NBUF >= 0)
                    def _():
                        write_wait(nb)

                    gather_start(cn, nb)

            return carry

        lax.fori_loop(0, nchunk // _NBUF, body, 0)
        for b in range(_NBUF):
            write_wait(b)

    return k


def kernel(obs, emb_table):
    batch, fields = obs.shape
    n_cat, d = emb_table.shape
    obs_t = obs.astype(jnp.int32).T
    out_t = _make_gather(batch, fields, d)(obs_t, emb_table)
    return jnp.transpose(out_t, (1, 0, 2))


# speedup vs baseline: 2.0815x; 1.0071x over previous
import functools

import jax
import jax.numpy as jnp
from jax import lax
from jax.experimental import pallas as pl
from jax.experimental.pallas import tpu as pltpu
from jax.experimental.pallas import tpu_sc as plsc

_NC = 2
_NS = 16
_NW = _NC * _NS
_C = 128
_NBUF = 6
_DEPTH = 5


@functools.lru_cache(maxsize=None)
def _make_gather(batch, fields, d):
    rows_w = batch // _NW
    nsub = rows_w // _C
    nchunk = fields * nsub
    mesh = plsc.VectorSubcoreMesh(core_axis_name="c", subcore_axis_name="s")

    @functools.partial(
        pl.kernel,
        mesh=mesh,
        out_type=jax.ShapeDtypeStruct((fields, batch, d), jnp.float32),
        scratch_types=[
            pltpu.VMEM((fields, rows_w), jnp.int32),
        ]
        + [pltpu.VMEM((_C, d), jnp.float32)] * _NBUF
        + [pltpu.SemaphoreType.DMA] * (2 * _NBUF),
    )
    def k(obs_t_hbm, table_hbm, out_hbm, idx_v, *rest):
        rows_v = rest[:_NBUF]
        sems = rest[_NBUF:]
        semg = sems[:_NBUF]
        semw = sems[_NBUF:]
        wid = lax.axis_index("s") * _NC + lax.axis_index("c")
        row0 = wid * rows_w
        pltpu.sync_copy(obs_t_hbm.at[:, pl.ds(row0, rows_w)], idx_v)

        def gather_start(c, b):
            f = c // nsub
            sub = c - f * nsub
            pltpu.make_async_copy(
                table_hbm.at[idx_v.at[f, pl.ds(sub * _C, _C)]],
                rows_v[b],
                semg[b],
            ).start()

        def gather_wait(b):
            pltpu.make_async_copy(
                table_hbm.at[idx_v.at[0, pl.ds(0, _C)]], rows_v[b], semg[b]
            ).wait()

        def write_start(c, b):
            f = c // nsub
            sub = c - f * nsub
            pltpu.make_async_copy(
                rows_v[b],
                out_hbm.at[f, pl.ds(row0 + sub * _C, _C)],
                semw[b],
            ).start()

        def write_wait(b):
            pltpu.make_async_copy(
                rows_v[b], out_hbm.at[0, pl.ds(row0, _C)], semw[b]
            ).wait()

        for c in range(_DEPTH):
            gather_start(c, c)

        def body(i, carry):
            for b in range(_NBUF):
                c = _NBUF * i + b
                gather_wait(b)
                write_start(c, b)
                nb = (b + _DEPTH) % _NBUF
                cn = c + _DEPTH

                @pl.when(cn < nchunk)
                def _():
                    @pl.when(cn - _NBUF >= 0)
                    def _():
                        write_wait(nb)

                    gather_start(cn, nb)

            return carry

        lax.fori_loop(0, nchunk // _NBUF, body, 0)
        for c in range(nchunk - nchunk % _NBUF, nchunk):
            gather_wait(c % _NBUF)
            write_start(c, c % _NBUF)
        for b in range(_NBUF):
            write_wait(b)

    return k


def kernel(obs, emb_table):
    batch, fields = obs.shape
    n_cat, d = emb_table.shape
    obs_t = obs.astype(jnp.int32).T
    out_t = _make_gather(batch, fields, d)(obs_t, emb_table)
    return jnp.transpose(out_t, (1, 0, 2))
